# baseline (speedup 1.0000x reference)
import functools

import jax
import jax.numpy as jnp
from jax import lax
from jax.experimental import pallas as pl
from jax.experimental.pallas import tpu as pltpu
from jax.experimental.pallas import tpu_sc as plsc

_pallas_call = pl.pallas_call

T = 2048
H = 2048
F = 1408
FS = 2816
E = 8
BT = 128
G = T * 2 // BT + E
NPAD = G * BT
FT = 704
NF = F // FT
FST = 256
NFS = FS // FST
WSC = 16
WG = 8


def _silu(v):
    return v / (1.0 + jnp.exp(-v))


def _routing_kernel(x_ref, wg_ref, tri_ref, p1_ref, p2_ref, w1_ref, w2_ref, be_ref):
    x = x_ref[...]
    logits = lax.dot_general(x, wg_ref[...], (((1,), (1,)), ((), ())),
                             preferred_element_type=jnp.float32)
    m = jnp.max(logits, axis=1, keepdims=True)
    ex = jnp.exp(logits - m)
    scores = ex / jnp.sum(ex, axis=1, keepdims=True)

    ii = lax.broadcasted_iota(jnp.int32, (T, E), 1)
    m1 = jnp.max(scores, axis=1, keepdims=True)
    idx1 = jnp.min(jnp.where(scores == m1, ii, E), axis=1, keepdims=True)
    oh1 = ii == idx1
    s2 = jnp.where(oh1, -1.0, scores)
    m2 = jnp.max(s2, axis=1, keepdims=True)
    idx2 = jnp.min(jnp.where(s2 == m2, ii, E), axis=1, keepdims=True)
    oh2 = ii == idx2

    sel = (oh1 | oh2).astype(jnp.bfloat16)
    csel = lax.dot_general(tri_ref[...], sel, (((1,), (0,)), ((), ())),
                           preferred_element_type=jnp.float32)
    counts = csel[T - 1:T, :]
    bcount = jnp.ceil(counts * (1.0 / BT)) * BT
    eiota_r = lax.broadcasted_iota(jnp.int32, (E, E), 0)
    eiota_c = lax.broadcasted_iota(jnp.int32, (E, E), 1)
    strict = (eiota_r < eiota_c).astype(jnp.bfloat16)
    offs = lax.dot_general(bcount.astype(jnp.bfloat16), strict,
                           (((1,), (0,)), ((), ())),
                           preferred_element_type=jnp.float32)
    posf = offs + csel - 1.0
    p1_ref[...] = jnp.sum(jnp.where(oh1, posf, 0.0), axis=1,
                          keepdims=True).astype(jnp.int32)
    p2_ref[...] = jnp.sum(jnp.where(oh2, posf, 0.0), axis=1,
                          keepdims=True).astype(jnp.int32)
    w1_ref[...] = m1
    w2_ref[...] = m2

    total = jnp.sum(bcount, axis=1, keepdims=True)
    gpos = (lax.broadcasted_iota(jnp.int32, (G, 1), 0) * BT).astype(jnp.float32)
    cnt = jnp.sum((jnp.broadcast_to(offs, (G, E)) <= gpos).astype(jnp.int32),
                  axis=1, keepdims=True)
    is_pad = (gpos >= total).astype(jnp.int32)
    be_ref[...] = cnt - 1 + 8 * is_pad


def _routing_call(xf, W_g, tri):
    return _pallas_call(
        _routing_kernel,
        out_shape=[
            jax.ShapeDtypeStruct((T, 1), jnp.int32),
            jax.ShapeDtypeStruct((T, 1), jnp.int32),
            jax.ShapeDtypeStruct((T, 1), jnp.float32),
            jax.ShapeDtypeStruct((T, 1), jnp.float32),
            jax.ShapeDtypeStruct((G, 1), jnp.int32),
        ],
    )(xf, W_g, tri)


def _dispatch_call(xf, p1r, p2r):
    mesh = plsc.VectorSubcoreMesh(core_axis_name="c", subcore_axis_name="s")
    NC, NS = mesh.num_cores, mesh.num_subcores
    NW = NC * NS
    PER_W = T // NW
    NSTEP = PER_W // WSC

    @functools.partial(
        pl.kernel, mesh=mesh,
        out_type=jax.ShapeDtypeStruct((NPAD, H), jnp.float32),
        scratch_types=[
            pltpu.VMEM((WSC,), jnp.int32),
            pltpu.VMEM((WSC,), jnp.int32),
            pltpu.VMEM((WSC, H), jnp.float32),
            pltpu.SemaphoreType.DMA,
        ])
    def k(x_hbm, p1_hbm, p2_hbm, xs_hbm, idx1_v, idx2_v, rows_v, sem):
        wid = lax.axis_index("s") * NC + lax.axis_index("c")
        for j in range(NSTEP):
            base = wid * PER_W + j * WSC
            pltpu.sync_copy(p1_hbm.at[pl.ds(base, WSC)], idx1_v)
            pltpu.sync_copy(p2_hbm.at[pl.ds(base, WSC)], idx2_v)
            pltpu.sync_copy(x_hbm.at[pl.ds(base, WSC)], rows_v)
            pltpu.async_copy(rows_v, xs_hbm.at[idx1_v], sem).wait()
            pltpu.async_copy(rows_v, xs_hbm.at[idx2_v], sem).wait()

    return k(xf, p1r, p2r)


def _ffn_kernel(be_ref, xs_ref, wg_ref, wu_ref, wd_ref, out_ref):
    g = pl.program_id(0)
    f = pl.program_id(1)
    active = be_ref[g] < 8

    @pl.when(active)
    def _():
        xb = xs_ref[...]
        gt = lax.dot_general(xb, wg_ref[...], (((1,), (1,)), ((), ())),
                             preferred_element_type=jnp.float32)
        up = lax.dot_general(xb, wu_ref[...], (((1,), (1,)), ((), ())),
                             preferred_element_type=jnp.float32)
        h = _silu(gt) * up
        wd = wd_ref[...].reshape(H, FT)
        contrib = lax.dot_general(h, wd, (((1,), (1,)), ((), ())),
                                  preferred_element_type=jnp.float32)
        out_ref[...] = jnp.where(f == 0, contrib, out_ref[...] + contrib)


def _ffn_call(be, xs, We_gate, We_up, We_down):
    wg_r = We_gate.reshape(E, NF, FT, H)
    wu_r = We_up.reshape(E, NF, FT, H)
    wd_r = We_down.reshape(E, H, NF, 1, FT)
    grid_spec = pltpu.PrefetchScalarGridSpec(
        num_scalar_prefetch=1,
        grid=(G, NF),
        in_specs=[
            pl.BlockSpec((BT, H), lambda g, f, be: (g, 0)),
            pl.BlockSpec((None, None, FT, H), lambda g, f, be: (be[g] % 8, f, 0, 0)),
            pl.BlockSpec((None, None, FT, H), lambda g, f, be: (be[g] % 8, f, 0, 0)),
            pl.BlockSpec((None, H, None, 1, FT),
                         lambda g, f, be: (be[g] % 8, 0, f, 0, 0)),
        ],
        out_specs=pl.BlockSpec((BT, H), lambda g, f, be: (g, 0)),
    )
    return _pallas_call(
        _ffn_kernel,
        grid_spec=grid_spec,
        out_shape=jax.ShapeDtypeStruct((NPAD, H), jnp.float32),
    )(be, xs, wg_r, wu_r, wd_r)


def _shared_kernel(x_ref, wg_ref, wu_ref, wd_ref, out_ref):
    f = pl.program_id(0)

    @pl.when(f == 0)
    def _():
        out_ref[...] = jnp.zeros_like(out_ref)

    x = x_ref[...]
    gt = lax.dot_general(x, wg_ref[...], (((1,), (1,)), ((), ())),
                         preferred_element_type=jnp.float32)
    up = lax.dot_general(x, wu_ref[...], (((1,), (1,)), ((), ())),
                         preferred_element_type=jnp.float32)
    h = _silu(gt) * up
    out_ref[...] += lax.dot_general(h, wd_ref[...], (((1,), (1,)), ((), ())),
                                    preferred_element_type=jnp.float32)


def _shared_call(xf, Ws_gate, Ws_up, Ws_down):
    return _pallas_call(
        _shared_kernel,
        grid=(NFS,),
        in_specs=[
            pl.BlockSpec((T, H), lambda f: (0, 0)),
            pl.BlockSpec((FST, H), lambda f: (f, 0)),
            pl.BlockSpec((FST, H), lambda f: (f, 0)),
            pl.BlockSpec((H, FST), lambda f: (0, f)),
        ],
        out_specs=pl.BlockSpec((T, H), lambda f: (0, 0)),
        out_shape=jax.ShapeDtypeStruct((T, H), jnp.float32),
    )(xf, Ws_gate, Ws_up, Ws_down)


def _gather_call(ys, p1r, p2r):
    mesh = plsc.VectorSubcoreMesh(core_axis_name="c", subcore_axis_name="s")
    NC, NS = mesh.num_cores, mesh.num_subcores
    NW = NC * NS
    PER_W = T // NW
    NSTEP = PER_W // WSC

    @functools.partial(
        pl.kernel, mesh=mesh,
        out_type=[jax.ShapeDtypeStruct((T, H), jnp.float32),
                  jax.ShapeDtypeStruct((T, H), jnp.float32)],
        scratch_types=[
            pltpu.VMEM((WSC,), jnp.int32),
            pltpu.VMEM((WSC,), jnp.int32),
            pltpu.VMEM((WSC, H), jnp.float32),
            pltpu.VMEM((WSC, H), jnp.float32),
            pltpu.SemaphoreType.DMA,
        ])
    def k(ys_hbm, p1_hbm, p2_hbm, g1_hbm, g2_hbm, idx1_v, idx2_v, b1_v, b2_v, sem):
        wid = lax.axis_index("s") * NC + lax.axis_index("c")
        for j in range(NSTEP):
            base = wid * PER_W + j * WSC
            pltpu.sync_copy(p1_hbm.at[pl.ds(base, WSC)], idx1_v)
            pltpu.sync_copy(p2_hbm.at[pl.ds(base, WSC)], idx2_v)
            pltpu.async_copy(ys_hbm.at[idx1_v], b1_v, sem).wait()
            pltpu.async_copy(ys_hbm.at[idx2_v], b2_v, sem).wait()
            pltpu.sync_copy(b1_v, g1_hbm.at[pl.ds(base, WSC)])
            pltpu.sync_copy(b2_v, g2_hbm.at[pl.ds(base, WSC)])

    return k(ys, p1r, p2r)


def _add3_kernel(sh_ref, g1_ref, g2_ref, w1_ref, w2_ref, out_ref):
    out_ref[...] = (sh_ref[...] + w1_ref[...] * g1_ref[...]
                    + w2_ref[...] * g2_ref[...])


def _add3_call(sh, g1, g2, w1, w2):
    TB = 256
    return _pallas_call(
        _add3_kernel,
        grid=(T // TB,),
        in_specs=[
            pl.BlockSpec((TB, H), lambda i: (i, 0)),
            pl.BlockSpec((TB, H), lambda i: (i, 0)),
            pl.BlockSpec((TB, H), lambda i: (i, 0)),
            pl.BlockSpec((TB, 1), lambda i: (i, 0)),
            pl.BlockSpec((TB, 1), lambda i: (i, 0)),
        ],
        out_specs=pl.BlockSpec((TB, H), lambda i: (i, 0)),
        out_shape=jax.ShapeDtypeStruct((T, H), jnp.float32),
    )(sh, g1, g2, w1, w2)


def kernel(x, W_g, We_gate, We_up, We_down, Ws_gate, Ws_up, Ws_down):
    xf = x.reshape(T, H)
    row = lax.broadcasted_iota(jnp.int32, (T, T), 0)
    col = lax.broadcasted_iota(jnp.int32, (T, T), 1)
    tri = (row >= col).astype(jnp.bfloat16)

    p1, p2, w1, w2, be = _routing_call(xf, W_g, tri)
    p1r = p1.reshape(T)
    p2r = p2.reshape(T)

    xs = _dispatch_call(xf, p1r, p2r)
    ys = _ffn_call(be.reshape(G), xs, We_gate, We_up, We_down)
    sh = _shared_call(xf, Ws_gate, Ws_up, Ws_down)
    g1, g2 = _gather_call(ys, p1r, p2r)
    out = _add3_call(sh, g1, g2, w1, w2)
    return out.reshape(x.shape)


# speedup vs baseline: 2.3225x; 2.3225x over previous
import functools

import jax
import jax.numpy as jnp
from jax import lax
from jax.experimental import pallas as pl
from jax.experimental.pallas import tpu as pltpu
from jax.experimental.pallas import tpu_sc as plsc

_pallas_call = pl.pallas_call

T = 2048
H = 2048
F = 1408
FS = 2816
E = 8
BT = 512
G = T * 2 // BT + E
NPAD = G * BT
NK = 4
HK = H // NK
FST = 256
NFS = FS // FST
WSC = 16
WG = 8


def _silu(v):
    return v / (1.0 + jnp.exp(-v))


def _dot_nt(a, b):
    return lax.dot_general(a, b, (((1,), (1,)), ((), ())),
                           precision=lax.Precision.DEFAULT,
                           preferred_element_type=jnp.float32)


def _routing_kernel(x_ref, wg_ref, tri_ref, p1_ref, p2_ref, w1_ref, w2_ref, be_ref):
    x = x_ref[...]
    logits = lax.dot_general(x, wg_ref[...], (((1,), (1,)), ((), ())),
                             preferred_element_type=jnp.float32)
    m = jnp.max(logits, axis=1, keepdims=True)
    ex = jnp.exp(logits - m)
    scores = ex / jnp.sum(ex, axis=1, keepdims=True)

    ii = lax.broadcasted_iota(jnp.int32, (T, E), 1)
    m1 = jnp.max(scores, axis=1, keepdims=True)
    idx1 = jnp.min(jnp.where(scores == m1, ii, E), axis=1, keepdims=True)
    oh1 = ii == idx1
    s2 = jnp.where(oh1, -1.0, scores)
    m2 = jnp.max(s2, axis=1, keepdims=True)
    idx2 = jnp.min(jnp.where(s2 == m2, ii, E), axis=1, keepdims=True)
    oh2 = ii == idx2

    sel = (oh1 | oh2).astype(jnp.bfloat16)
    csel = lax.dot_general(tri_ref[...], sel, (((1,), (0,)), ((), ())),
                           preferred_element_type=jnp.float32)
    counts = csel[T - 1:T, :]
    bcount = jnp.ceil(counts * (1.0 / BT)) * BT
    eiota_r = lax.broadcasted_iota(jnp.int32, (E, E), 0)
    eiota_c = lax.broadcasted_iota(jnp.int32, (E, E), 1)
    strict = (eiota_r < eiota_c).astype(jnp.bfloat16)
    offs = lax.dot_general(bcount.astype(jnp.bfloat16), strict,
                           (((1,), (0,)), ((), ())),
                           preferred_element_type=jnp.float32)
    posf = offs + csel - 1.0
    p1_ref[...] = jnp.sum(jnp.where(oh1, posf, 0.0), axis=1,
                          keepdims=True).astype(jnp.int32)
    p2_ref[...] = jnp.sum(jnp.where(oh2, posf, 0.0), axis=1,
                          keepdims=True).astype(jnp.int32)
    w1_ref[...] = m1
    w2_ref[...] = m2

    total = jnp.sum(bcount, axis=1, keepdims=True)
    gpos = (lax.broadcasted_iota(jnp.int32, (G, 1), 0) * BT).astype(jnp.float32)
    cnt = jnp.sum((jnp.broadcast_to(offs, (G, E)) <= gpos).astype(jnp.int32),
                  axis=1, keepdims=True)
    is_pad = (gpos >= total).astype(jnp.int32)
    be_ref[...] = cnt - 1 + 8 * is_pad


def _routing_call(xf, W_g, tri):
    return _pallas_call(
        _routing_kernel,
        out_shape=[
            jax.ShapeDtypeStruct((T, 1), jnp.int32),
            jax.ShapeDtypeStruct((T, 1), jnp.int32),
            jax.ShapeDtypeStruct((T, 1), jnp.float32),
            jax.ShapeDtypeStruct((T, 1), jnp.float32),
            jax.ShapeDtypeStruct((G, 1), jnp.int32),
        ],
    )(xf, W_g, tri)


def _dispatch_call(xf, p1r, p2r):
    mesh = plsc.VectorSubcoreMesh(core_axis_name="c", subcore_axis_name="s")
    NC, NS = mesh.num_cores, mesh.num_subcores
    NW = NC * NS
    PER_W = T // NW
    NSTEP = PER_W // WSC

    @functools.partial(
        pl.kernel, mesh=mesh,
        out_type=jax.ShapeDtypeStruct((NPAD, H), jnp.float32),
        scratch_types=[
            pltpu.VMEM((WSC,), jnp.int32),
            pltpu.VMEM((WSC,), jnp.int32),
            pltpu.VMEM((WSC, H), jnp.float32),
            pltpu.SemaphoreType.DMA,
        ])
    def k(x_hbm, p1_hbm, p2_hbm, xs_hbm, idx1_v, idx2_v, rows_v, sem):
        wid = lax.axis_index("s") * NC + lax.axis_index("c")
        for j in range(NSTEP):
            base = wid * PER_W + j * WSC
            pltpu.sync_copy(p1_hbm.at[pl.ds(base, WSC)], idx1_v)
            pltpu.sync_copy(p2_hbm.at[pl.ds(base, WSC)], idx2_v)
            pltpu.sync_copy(x_hbm.at[pl.ds(base, WSC)], rows_v)
            pltpu.async_copy(rows_v, xs_hbm.at[idx1_v], sem).wait()
            pltpu.async_copy(rows_v, xs_hbm.at[idx2_v], sem).wait()

    return k(xf, p1r, p2r)


def _ffn_kernel(be_ref, xs_ref, wg_ref, wu_ref, wd_ref, out_ref, gt_ref, up_ref):
    g = pl.program_id(0)
    k = pl.program_id(1)
    active = be_ref[g] < 8

    @pl.when(active)
    def _():
        xb = xs_ref[...]
        pg = _dot_nt(xb, wg_ref[...])
        pu = _dot_nt(xb, wu_ref[...])
        gt_ref[...] = jnp.where(k == 0, pg, gt_ref[...] + pg)
        up_ref[...] = jnp.where(k == 0, pu, up_ref[...] + pu)

    @pl.when(active & (k == NK - 1))
    def _():
        h = _silu(gt_ref[...]) * up_ref[...]
        out_ref[...] = _dot_nt(h, wd_ref[...])


def _ffn_call(be, xs, We_gate, We_up, We_down):
    def kchunk(g, k, be):
        snake = jnp.where(g % 2 == 0, k, NK - 1 - k)
        return jnp.where(be[g] >= 8, 0, snake)

    def wmap(g, k, be):
        return (be[g] % 8, 0, kchunk(g, k, be))

    grid_spec = pltpu.PrefetchScalarGridSpec(
        num_scalar_prefetch=1,
        grid=(G, NK),
        in_specs=[
            pl.BlockSpec((BT, HK), lambda g, k, be: (g, kchunk(g, k, be))),
            pl.BlockSpec((None, F, HK), wmap),
            pl.BlockSpec((None, F, HK), wmap),
            pl.BlockSpec((None, H, F), lambda g, k, be: (be[g] % 8, 0, 0)),
        ],
        out_specs=pl.BlockSpec((BT, H), lambda g, k, be: (g, 0)),
        scratch_shapes=[pltpu.VMEM((BT, F), jnp.float32),
                        pltpu.VMEM((BT, F), jnp.float32)],
    )
    return _pallas_call(
        _ffn_kernel,
        grid_spec=grid_spec,
        out_shape=jax.ShapeDtypeStruct((NPAD, H), jnp.float32),
    )(be, xs, We_gate, We_up, We_down)


def _shared_kernel(x_ref, wg_ref, wu_ref, wd_ref, out_ref):
    f = pl.program_id(0)

    @pl.when(f == 0)
    def _():
        out_ref[...] = jnp.zeros_like(out_ref)

    x = x_ref[...]
    gt = _dot_nt(x, wg_ref[...])
    up = _dot_nt(x, wu_ref[...])
    h = _silu(gt) * up
    out_ref[...] += _dot_nt(h, wd_ref[...])


def _shared_call(xf, Ws_gate, Ws_up, Ws_down):
    return _pallas_call(
        _shared_kernel,
        grid=(NFS,),
        in_specs=[
            pl.BlockSpec((T, H), lambda f: (0, 0)),
            pl.BlockSpec((FST, H), lambda f: (f, 0)),
            pl.BlockSpec((FST, H), lambda f: (f, 0)),
            pl.BlockSpec((H, FST), lambda f: (0, f)),
        ],
        out_specs=pl.BlockSpec((T, H), lambda f: (0, 0)),
        out_shape=jax.ShapeDtypeStruct((T, H), jnp.float32),
    )(xf, Ws_gate, Ws_up, Ws_down)


def _gather_call(ys, p1r, p2r):
    mesh = plsc.VectorSubcoreMesh(core_axis_name="c", subcore_axis_name="s")
    NC, NS = mesh.num_cores, mesh.num_subcores
    NW = NC * NS
    PER_W = T // NW
    NSTEP = PER_W // WSC

    @functools.partial(
        pl.kernel, mesh=mesh,
        out_type=[jax.ShapeDtypeStruct((T, H), jnp.float32),
                  jax.ShapeDtypeStruct((T, H), jnp.float32)],
        scratch_types=[
            pltpu.VMEM((WSC,), jnp.int32),
            pltpu.VMEM((WSC,), jnp.int32),
            pltpu.VMEM((WSC, H), jnp.float32),
            pltpu.VMEM((WSC, H), jnp.float32),
            pltpu.SemaphoreType.DMA,
        ])
    def k(ys_hbm, p1_hbm, p2_hbm, g1_hbm, g2_hbm, idx1_v, idx2_v, b1_v, b2_v, sem):
        wid = lax.axis_index("s") * NC + lax.axis_index("c")
        for j in range(NSTEP):
            base = wid * PER_W + j * WSC
            pltpu.sync_copy(p1_hbm.at[pl.ds(base, WSC)], idx1_v)
            pltpu.sync_copy(p2_hbm.at[pl.ds(base, WSC)], idx2_v)
            pltpu.async_copy(ys_hbm.at[idx1_v], b1_v, sem).wait()
            pltpu.async_copy(ys_hbm.at[idx2_v], b2_v, sem).wait()
            pltpu.sync_copy(b1_v, g1_hbm.at[pl.ds(base, WSC)])
            pltpu.sync_copy(b2_v, g2_hbm.at[pl.ds(base, WSC)])

    return k(ys, p1r, p2r)


def _add3_kernel(sh_ref, g1_ref, g2_ref, w1_ref, w2_ref, out_ref):
    out_ref[...] = (sh_ref[...] + w1_ref[...] * g1_ref[...]
                    + w2_ref[...] * g2_ref[...])


def _add3_call(sh, g1, g2, w1, w2):
    TB = 256
    return _pallas_call(
        _add3_kernel,
        grid=(T // TB,),
        in_specs=[
            pl.BlockSpec((TB, H), lambda i: (i, 0)),
            pl.BlockSpec((TB, H), lambda i: (i, 0)),
            pl.BlockSpec((TB, H), lambda i: (i, 0)),
            pl.BlockSpec((TB, 1), lambda i: (i, 0)),
            pl.BlockSpec((TB, 1), lambda i: (i, 0)),
        ],
        out_specs=pl.BlockSpec((TB, H), lambda i: (i, 0)),
        out_shape=jax.ShapeDtypeStruct((T, H), jnp.float32),
    )(sh, g1, g2, w1, w2)


def kernel(x, W_g, We_gate, We_up, We_down, Ws_gate, Ws_up, Ws_down):
    xf = x.reshape(T, H)
    row = lax.broadcasted_iota(jnp.int32, (T, T), 0)
    col = lax.broadcasted_iota(jnp.int32, (T, T), 1)
    tri = (row >= col).astype(jnp.bfloat16)

    p1, p2, w1, w2, be = _routing_call(xf, W_g, tri)
    p1r = p1.reshape(T)
    p2r = p2.reshape(T)

    xs = _dispatch_call(xf, p1r, p2r)
    ys = _ffn_call(be.reshape(G), xs, We_gate, We_up, We_down)
    sh = _shared_call(xf, Ws_gate, Ws_up, Ws_down)
    g1, g2 = _gather_call(ys, p1r, p2r)
    out = _add3_call(sh, g1, g2, w1, w2)
    return out.reshape(x.shape)
